# baseline (speedup 1.0000x reference)
import functools

import jax
import jax.numpy as jnp
from jax.experimental import pallas as pl

EPS = 1e-5


def _fps_body(x, iota, iota_np, n, i, state):
    cent, dist, far = state
    cent = jnp.where(iota_np == i, far, cent)
    mask = iota == far
    c = jnp.sum(jnp.where(mask, x, 0.0), axis=1, keepdims=True)
    d = jnp.sum((x - c) ** 2, axis=0, keepdims=True)
    dist = jnp.minimum(dist, d)
    mx = jnp.max(dist)
    far = jnp.min(jnp.where(dist == mx, iota, n)).astype(jnp.int32)
    return cent, dist, far


def _fps_kernel(xyz_ref, out_ref, *, npoint):
    x = xyz_ref[0]
    n = x.shape[1]
    iota = jax.lax.broadcasted_iota(jnp.int32, (1, n), 1)
    iota_np = jax.lax.broadcasted_iota(jnp.int32, (1, npoint), 1)
    init = (
        jnp.zeros((1, npoint), jnp.int32),
        jnp.full((1, n), 1e10, jnp.float32),
        jnp.zeros((), jnp.int32),
    )
    cent, _, _ = jax.lax.fori_loop(
        0, npoint, functools.partial(_fps_body, x, iota, iota_np, n), init
    )
    out_ref[...] = jnp.broadcast_to(cent, out_ref.shape)


def _fps(xyz, npoint):
    b, n, _ = xyz.shape
    xt = jnp.transpose(xyz, (0, 2, 1))
    out = pl.pallas_call(
        functools.partial(_fps_kernel, npoint=npoint),
        grid=(b,),
        in_specs=[pl.BlockSpec((1, 3, n), lambda i: (i, 0, 0))],
        out_specs=pl.BlockSpec((1, 8, npoint), lambda i: (i, 0, 0)),
        out_shape=jax.ShapeDtypeStruct((b, 8, npoint), jnp.int32),
    )(xt)
    return out[:, 0, :]


def _mm_kernel(x_ref, w_ref, b_ref, o_ref):
    o_ref[...] = (
        jnp.dot(x_ref[...], w_ref[...].T, preferred_element_type=jnp.float32)
        + b_ref[...]
    )


def _mm_act_kernel(x_ref, w_ref, b_ref, s_ref, t_ref, o_ref):
    x = jnp.maximum(x_ref[...] * s_ref[...] + t_ref[...], 0.0)
    o_ref[...] = (
        jnp.dot(x, w_ref[...].T, preferred_element_type=jnp.float32) + b_ref[...]
    )


def _matmul(x, w, bvec, scale=None, shift=None):
    m, c = x.shape
    o = w.shape[0]
    tm = min(m, 2048)
    grid = (m // tm,)
    b2 = bvec.reshape(1, o)
    x_spec = pl.BlockSpec((tm, c), lambda i: (i, 0))
    w_spec = pl.BlockSpec((o, c), lambda i: (0, 0))
    b_spec = pl.BlockSpec((1, o), lambda i: (0, 0))
    v_spec = pl.BlockSpec((1, c), lambda i: (0, 0))
    out_spec = pl.BlockSpec((tm, o), lambda i: (i, 0))
    out_shape = jax.ShapeDtypeStruct((m, o), jnp.float32)
    if scale is None:
        return pl.pallas_call(
            _mm_kernel,
            grid=grid,
            in_specs=[x_spec, w_spec, b_spec],
            out_specs=out_spec,
            out_shape=out_shape,
        )(x, w, b2)
    return pl.pallas_call(
        _mm_act_kernel,
        grid=grid,
        in_specs=[x_spec, w_spec, b_spec, v_spec, v_spec],
        out_specs=out_spec,
        out_shape=out_shape,
    )(x, w, b2, scale.reshape(1, c), shift.reshape(1, c))


def _arm_kernel(y_ref, s_ref, t_ref, o_ref):
    o_ref[...] = jnp.max(
        jnp.maximum(y_ref[...] * s_ref[...] + t_ref[...], 0.0), axis=1
    )


def _affine_relu_max(y, scale, shift):
    m2, k, o = y.shape
    ts = min(m2, 8)
    return pl.pallas_call(
        _arm_kernel,
        grid=(m2 // ts,),
        in_specs=[
            pl.BlockSpec((ts, k, o), lambda i: (i, 0, 0)),
            pl.BlockSpec((1, 1, o), lambda i: (0, 0, 0)),
            pl.BlockSpec((1, 1, o), lambda i: (0, 0, 0)),
        ],
        out_specs=pl.BlockSpec((ts, o), lambda i: (i, 0)),
        out_shape=jax.ShapeDtypeStruct((m2, o), jnp.float32),
    )(y, scale.reshape(1, 1, o), shift.reshape(1, 1, o))


def _bn_affine(y, layer):
    mean = jnp.mean(y, axis=0)
    var = jnp.var(y, axis=0)
    scale = layer["gamma"] / jnp.sqrt(var + EPS)
    shift = layer["beta"] - mean * scale
    return scale, shift


def _mlp_max(grouped, layers):
    b, s, k, c = grouped.shape
    x = grouped.reshape(b * s * k, c)
    scale = shift = None
    for layer in layers:
        x = _matmul(x, layer["W"], layer["b"], scale, shift)
        scale, shift = _bn_affine(x, layer)
    o = x.shape[-1]
    out = _affine_relu_max(x.reshape(b * s, k, o), scale, shift)
    return out.reshape(b, s, o)


def _gather(points, idx):
    b = points.shape[0]
    batch_idx = jnp.arange(b).reshape((b,) + (1,) * (idx.ndim - 1))
    return points[batch_idx, idx]


def _ball_query(radius, nsample, xyz, new_xyz):
    b, n, _ = xyz.shape
    s = new_xyz.shape[1]
    sqrdists = (
        jnp.sum(new_xyz**2, axis=-1)[:, :, None]
        + jnp.sum(xyz**2, axis=-1)[:, None, :]
        - 2.0 * jnp.matmul(new_xyz, jnp.transpose(xyz, (0, 2, 1)))
    )
    group_idx = jnp.broadcast_to(jnp.arange(n, dtype=jnp.int32), (b, s, n))
    group_idx = jnp.where(sqrdists > radius**2, n, group_idx)
    group_idx = jnp.sort(group_idx, axis=-1)[:, :, :nsample]
    group_first = jnp.broadcast_to(group_idx[:, :, :1], group_idx.shape)
    return jnp.where(group_idx == n, group_first, group_idx)


def _sa_msg(xyz, points, npoint, radius_list, nsample_list, branches):
    fps_idx = _fps(xyz, npoint)
    new_xyz = _gather(xyz, fps_idx)
    outs = []
    for radius, nsample, layers in zip(radius_list, nsample_list, branches):
        group_idx = _ball_query(radius, nsample, xyz, new_xyz)
        grouped_xyz = _gather(xyz, group_idx) - new_xyz[:, :, None, :]
        if points is not None:
            grouped = jnp.concatenate(
                [_gather(points, group_idx), grouped_xyz], axis=-1
            )
        else:
            grouped = grouped_xyz
        outs.append(_mlp_max(grouped, layers))
    return new_xyz, jnp.concatenate(outs, axis=-1)


def kernel(xyz, params):
    pts = jnp.transpose(xyz, (0, 2, 1))
    l1_xyz, l1_points = _sa_msg(
        pts, None, 512, [0.1, 0.2, 0.4], [16, 32, 128], params["sa1"]
    )
    l2_xyz, l2_points = _sa_msg(
        l1_xyz, l1_points, 128, [0.2, 0.4, 0.8], [32, 64, 128], params["sa2"]
    )
    grouped = jnp.concatenate([l2_xyz, l2_points], axis=-1)[:, None, :, :]
    l3 = _mlp_max(grouped, params["sa3"])
    return l3.reshape(l3.shape[0], 1024)


# speedup vs baseline: 1.0601x; 1.0601x over previous
import functools

import jax
import jax.numpy as jnp
from jax.experimental import pallas as pl

EPS = 1e-5


def _fps_body(x, iota, iota_np, n, i, state):
    cent, dist, far = state
    cent = jnp.where(iota_np == i, far, cent)
    mask = iota == far
    c = jnp.sum(jnp.where(mask, x, 0.0), axis=2, keepdims=True)
    d = jnp.sum((x - c) ** 2, axis=1, keepdims=True)
    dist = jnp.minimum(dist, d)
    mx = jnp.max(dist, axis=2, keepdims=True)
    far = jnp.min(jnp.where(dist == mx, iota, n), axis=2, keepdims=True)
    return cent, dist, far


def _fps_kernel(xyz_ref, out_ref, *, npoint):
    x = xyz_ref[...]
    b, _, n = x.shape
    iota = jax.lax.broadcasted_iota(jnp.int32, (1, 1, n), 2)
    iota_np = jax.lax.broadcasted_iota(jnp.int32, (1, 1, npoint), 2)
    init = (
        jnp.zeros((b, 1, npoint), jnp.int32),
        jnp.full((b, 1, n), 1e10, jnp.float32),
        jnp.zeros((b, 1, 1), jnp.int32),
    )
    cent, _, _ = jax.lax.fori_loop(
        0, npoint, functools.partial(_fps_body, x, iota, iota_np, n), init
    )
    out_ref[...] = cent.reshape(b, npoint)


def _fps(xyz, npoint):
    b, n, _ = xyz.shape
    xt = jnp.transpose(xyz, (0, 2, 1))
    return pl.pallas_call(
        functools.partial(_fps_kernel, npoint=npoint),
        in_specs=[pl.BlockSpec((b, 3, n), lambda: (0, 0, 0))],
        out_specs=pl.BlockSpec((b, npoint), lambda: (0, 0)),
        out_shape=jax.ShapeDtypeStruct((b, npoint), jnp.int32),
    )(xt)


def _mm_kernel(x_ref, w_ref, b_ref, o_ref):
    o_ref[...] = (
        jnp.dot(x_ref[...], w_ref[...].T, preferred_element_type=jnp.float32)
        + b_ref[...]
    )


def _mm_act_kernel(x_ref, w_ref, b_ref, s_ref, t_ref, o_ref):
    x = jnp.maximum(x_ref[...] * s_ref[...] + t_ref[...], 0.0)
    o_ref[...] = (
        jnp.dot(x, w_ref[...].T, preferred_element_type=jnp.float32) + b_ref[...]
    )


def _matmul(x, w, bvec, scale=None, shift=None):
    m, c = x.shape
    o = w.shape[0]
    tm = min(m, 2048)
    grid = (m // tm,)
    b2 = bvec.reshape(1, o)
    x_spec = pl.BlockSpec((tm, c), lambda i: (i, 0))
    w_spec = pl.BlockSpec((o, c), lambda i: (0, 0))
    b_spec = pl.BlockSpec((1, o), lambda i: (0, 0))
    v_spec = pl.BlockSpec((1, c), lambda i: (0, 0))
    out_spec = pl.BlockSpec((tm, o), lambda i: (i, 0))
    out_shape = jax.ShapeDtypeStruct((m, o), jnp.float32)
    if scale is None:
        return pl.pallas_call(
            _mm_kernel,
            grid=grid,
            in_specs=[x_spec, w_spec, b_spec],
            out_specs=out_spec,
            out_shape=out_shape,
        )(x, w, b2)
    return pl.pallas_call(
        _mm_act_kernel,
        grid=grid,
        in_specs=[x_spec, w_spec, b_spec, v_spec, v_spec],
        out_specs=out_spec,
        out_shape=out_shape,
    )(x, w, b2, scale.reshape(1, c), shift.reshape(1, c))


def _arm_kernel(y_ref, s_ref, t_ref, o_ref):
    o_ref[...] = jnp.max(
        jnp.maximum(y_ref[...] * s_ref[...] + t_ref[...], 0.0), axis=1
    )


def _affine_relu_max(y, scale, shift):
    m2, k, o = y.shape
    ts = min(m2, 8)
    return pl.pallas_call(
        _arm_kernel,
        grid=(m2 // ts,),
        in_specs=[
            pl.BlockSpec((ts, k, o), lambda i: (i, 0, 0)),
            pl.BlockSpec((1, 1, o), lambda i: (0, 0, 0)),
            pl.BlockSpec((1, 1, o), lambda i: (0, 0, 0)),
        ],
        out_specs=pl.BlockSpec((ts, o), lambda i: (i, 0)),
        out_shape=jax.ShapeDtypeStruct((m2, o), jnp.float32),
    )(y, scale.reshape(1, 1, o), shift.reshape(1, 1, o))


def _bn_affine(y, layer):
    mean = jnp.mean(y, axis=0)
    var = jnp.var(y, axis=0)
    scale = layer["gamma"] / jnp.sqrt(var + EPS)
    shift = layer["beta"] - mean * scale
    return scale, shift


def _mlp_max(grouped, layers):
    b, s, k, c = grouped.shape
    x = grouped.reshape(b * s * k, c)
    scale = shift = None
    for layer in layers:
        x = _matmul(x, layer["W"], layer["b"], scale, shift)
        scale, shift = _bn_affine(x, layer)
    o = x.shape[-1]
    out = _affine_relu_max(x.reshape(b * s, k, o), scale, shift)
    return out.reshape(b, s, o)


def _gather(points, idx):
    b = points.shape[0]
    batch_idx = jnp.arange(b).reshape((b,) + (1,) * (idx.ndim - 1))
    return points[batch_idx, idx]


def _ball_query(radius, nsample, xyz, new_xyz):
    b, n, _ = xyz.shape
    s = new_xyz.shape[1]
    sqrdists = (
        jnp.sum(new_xyz**2, axis=-1)[:, :, None]
        + jnp.sum(xyz**2, axis=-1)[:, None, :]
        - 2.0 * jnp.matmul(new_xyz, jnp.transpose(xyz, (0, 2, 1)))
    )
    group_idx = jnp.broadcast_to(jnp.arange(n, dtype=jnp.int32), (b, s, n))
    group_idx = jnp.where(sqrdists > radius**2, n, group_idx)
    group_idx = jnp.sort(group_idx, axis=-1)[:, :, :nsample]
    group_first = jnp.broadcast_to(group_idx[:, :, :1], group_idx.shape)
    return jnp.where(group_idx == n, group_first, group_idx)


def _sa_msg(xyz, points, npoint, radius_list, nsample_list, branches):
    fps_idx = _fps(xyz, npoint)
    new_xyz = _gather(xyz, fps_idx)
    outs = []
    for radius, nsample, layers in zip(radius_list, nsample_list, branches):
        group_idx = _ball_query(radius, nsample, xyz, new_xyz)
        grouped_xyz = _gather(xyz, group_idx) - new_xyz[:, :, None, :]
        if points is not None:
            grouped = jnp.concatenate(
                [_gather(points, group_idx), grouped_xyz], axis=-1
            )
        else:
            grouped = grouped_xyz
        outs.append(_mlp_max(grouped, layers))
    return new_xyz, jnp.concatenate(outs, axis=-1)


def kernel(xyz, params):
    pts = jnp.transpose(xyz, (0, 2, 1))
    l1_xyz, l1_points = _sa_msg(
        pts, None, 512, [0.1, 0.2, 0.4], [16, 32, 128], params["sa1"]
    )
    l2_xyz, l2_points = _sa_msg(
        l1_xyz, l1_points, 128, [0.2, 0.4, 0.8], [32, 64, 128], params["sa2"]
    )
    grouped = jnp.concatenate([l2_xyz, l2_points], axis=-1)[:, None, :, :]
    l3 = _mlp_max(grouped, params["sa3"])
    return l3.reshape(l3.shape[0], 1024)


# speedup vs baseline: 1.1081x; 1.0453x over previous
import functools

import jax
import jax.numpy as jnp
from jax.experimental import pallas as pl

EPS = 1e-5


def _fps_body(x, iota, iota_np, n, i, state):
    cent, dist, far = state
    cent = jnp.where(iota_np == i, far, cent)
    mask = iota == far
    c = jnp.sum(jnp.where(mask, x, 0.0), axis=2, keepdims=True)
    d = jnp.sum((x - c) ** 2, axis=1, keepdims=True)
    dist = jnp.minimum(dist, d)
    mx = jnp.max(dist, axis=2, keepdims=True)
    far = jnp.min(jnp.where(dist == mx, iota, n), axis=2, keepdims=True)
    return cent, dist, far


def _fps_kernel(xyz_ref, out_ref, *, npoint):
    x = xyz_ref[...]
    b, _, n = x.shape
    iota = jax.lax.broadcasted_iota(jnp.int32, (1, 1, n), 2)
    iota_np = jax.lax.broadcasted_iota(jnp.int32, (1, 1, npoint), 2)
    init = (
        jnp.zeros((b, 1, npoint), jnp.int32),
        jnp.full((b, 1, n), 1e10, jnp.float32),
        jnp.zeros((b, 1, 1), jnp.int32),
    )
    cent, _, _ = jax.lax.fori_loop(
        0, npoint, functools.partial(_fps_body, x, iota, iota_np, n), init
    )
    out_ref[...] = cent.reshape(b, npoint)


def _fps(xyz, npoint):
    b, n, _ = xyz.shape
    xt = jnp.transpose(xyz, (0, 2, 1))
    return pl.pallas_call(
        functools.partial(_fps_kernel, npoint=npoint),
        in_specs=[pl.BlockSpec((b, 3, n), lambda: (0, 0, 0))],
        out_specs=pl.BlockSpec((b, npoint), lambda: (0, 0)),
        out_shape=jax.ShapeDtypeStruct((b, npoint), jnp.int32),
    )(xt)


def _mm_kernel(x_ref, w_ref, b_ref, o_ref):
    o_ref[...] = (
        jnp.dot(x_ref[...], w_ref[...].T, preferred_element_type=jnp.float32)
        + b_ref[...]
    )


def _mm_act_kernel(x_ref, w_ref, b_ref, s_ref, t_ref, o_ref):
    x = jnp.maximum(x_ref[...] * s_ref[...] + t_ref[...], 0.0)
    o_ref[...] = (
        jnp.dot(x, w_ref[...].T, preferred_element_type=jnp.float32) + b_ref[...]
    )


def _matmul(x, w, bvec, scale=None, shift=None):
    m, c = x.shape
    o = w.shape[0]
    tm = min(m, 4096)
    grid = (m // tm,)
    b2 = bvec.reshape(1, o)
    x_spec = pl.BlockSpec((tm, c), lambda i: (i, 0))
    w_spec = pl.BlockSpec((o, c), lambda i: (0, 0))
    b_spec = pl.BlockSpec((1, o), lambda i: (0, 0))
    v_spec = pl.BlockSpec((1, c), lambda i: (0, 0))
    out_spec = pl.BlockSpec((tm, o), lambda i: (i, 0))
    out_shape = jax.ShapeDtypeStruct((m, o), jnp.float32)
    if scale is None:
        return pl.pallas_call(
            _mm_kernel,
            grid=grid,
            in_specs=[x_spec, w_spec, b_spec],
            out_specs=out_spec,
            out_shape=out_shape,
        )(x, w, b2)
    return pl.pallas_call(
        _mm_act_kernel,
        grid=grid,
        in_specs=[x_spec, w_spec, b_spec, v_spec, v_spec],
        out_specs=out_spec,
        out_shape=out_shape,
    )(x, w, b2, scale.reshape(1, c), shift.reshape(1, c))


def _arm_kernel(y_ref, s_ref, t_ref, o_ref):
    o_ref[...] = jnp.max(
        jnp.maximum(y_ref[...] * s_ref[...] + t_ref[...], 0.0), axis=1
    )


def _affine_relu_max(y, scale, shift):
    m2, k, o = y.shape
    ts = 8
    while ts * 2 <= m2 and m2 % (ts * 2) == 0 and ts * 2 * k * o * 4 <= 4 * 2**20:
        ts *= 2
    return pl.pallas_call(
        _arm_kernel,
        grid=(m2 // ts,),
        in_specs=[
            pl.BlockSpec((ts, k, o), lambda i: (i, 0, 0)),
            pl.BlockSpec((1, 1, o), lambda i: (0, 0, 0)),
            pl.BlockSpec((1, 1, o), lambda i: (0, 0, 0)),
        ],
        out_specs=pl.BlockSpec((ts, o), lambda i: (i, 0)),
        out_shape=jax.ShapeDtypeStruct((m2, o), jnp.float32),
    )(y, scale.reshape(1, 1, o), shift.reshape(1, 1, o))


def _bn_affine(y, layer):
    mean = jnp.mean(y, axis=0)
    var = jnp.var(y, axis=0)
    scale = layer["gamma"] / jnp.sqrt(var + EPS)
    shift = layer["beta"] - mean * scale
    return scale, shift


def _mlp_max(grouped, layers):
    b, s, k, c = grouped.shape
    x = grouped.reshape(b * s * k, c)
    scale = shift = None
    for layer in layers:
        x = _matmul(x, layer["W"], layer["b"], scale, shift)
        scale, shift = _bn_affine(x, layer)
    o = x.shape[-1]
    out = _affine_relu_max(x.reshape(b * s, k, o), scale, shift)
    return out.reshape(b, s, o)


def _gather(points, idx):
    b = points.shape[0]
    batch_idx = jnp.arange(b).reshape((b,) + (1,) * (idx.ndim - 1))
    return points[batch_idx, idx]


def _ball_query(radius, nsample, xyz, new_xyz):
    b, n, _ = xyz.shape
    s = new_xyz.shape[1]
    sqrdists = (
        jnp.sum(new_xyz**2, axis=-1)[:, :, None]
        + jnp.sum(xyz**2, axis=-1)[:, None, :]
        - 2.0 * jnp.matmul(new_xyz, jnp.transpose(xyz, (0, 2, 1)))
    )
    group_idx = jnp.broadcast_to(jnp.arange(n, dtype=jnp.int32), (b, s, n))
    group_idx = jnp.where(sqrdists > radius**2, n, group_idx)
    group_idx = jnp.sort(group_idx, axis=-1)[:, :, :nsample]
    group_first = jnp.broadcast_to(group_idx[:, :, :1], group_idx.shape)
    return jnp.where(group_idx == n, group_first, group_idx)


def _sa_msg(xyz, points, npoint, radius_list, nsample_list, branches):
    fps_idx = _fps(xyz, npoint)
    new_xyz = _gather(xyz, fps_idx)
    outs = []
    for radius, nsample, layers in zip(radius_list, nsample_list, branches):
        group_idx = _ball_query(radius, nsample, xyz, new_xyz)
        grouped_xyz = _gather(xyz, group_idx) - new_xyz[:, :, None, :]
        if points is not None:
            grouped = jnp.concatenate(
                [_gather(points, group_idx), grouped_xyz], axis=-1
            )
        else:
            grouped = grouped_xyz
        outs.append(_mlp_max(grouped, layers))
    return new_xyz, jnp.concatenate(outs, axis=-1)


def kernel(xyz, params):
    pts = jnp.transpose(xyz, (0, 2, 1))
    l1_xyz, l1_points = _sa_msg(
        pts, None, 512, [0.1, 0.2, 0.4], [16, 32, 128], params["sa1"]
    )
    l2_xyz, l2_points = _sa_msg(
        l1_xyz, l1_points, 128, [0.2, 0.4, 0.8], [32, 64, 128], params["sa2"]
    )
    grouped = jnp.concatenate([l2_xyz, l2_points], axis=-1)[:, None, :, :]
    l3 = _mlp_max(grouped, params["sa3"])
    return l3.reshape(l3.shape[0], 1024)


# speedup vs baseline: 1.1456x; 1.0339x over previous
import functools

import jax
import jax.numpy as jnp
from jax.experimental import pallas as pl

EPS = 1e-5


def _fps_body(x, iota, iota_np, n, i, state):
    cent, dist, far = state
    cent = jnp.where(iota_np == i, far, cent)
    mask = iota == far
    c = jnp.sum(jnp.where(mask, x, 0.0), axis=2, keepdims=True)
    d = jnp.sum((x - c) ** 2, axis=1, keepdims=True)
    dist = jnp.minimum(dist, d)
    mx = jnp.max(dist, axis=2, keepdims=True)
    far = jnp.min(jnp.where(dist == mx, iota, n), axis=2, keepdims=True)
    return cent, dist, far


def _fps_kernel(xyz_ref, out_ref, *, npoint):
    x = xyz_ref[...]
    b, _, n = x.shape
    iota = jax.lax.broadcasted_iota(jnp.int32, (1, 1, n), 2)
    iota_np = jax.lax.broadcasted_iota(jnp.int32, (1, 1, npoint), 2)
    init = (
        jnp.zeros((b, 1, npoint), jnp.int32),
        jnp.full((b, 1, n), 1e10, jnp.float32),
        jnp.zeros((b, 1, 1), jnp.int32),
    )
    cent, _, _ = jax.lax.fori_loop(
        0, npoint, functools.partial(_fps_body, x, iota, iota_np, n), init
    )
    out_ref[...] = cent.reshape(b, npoint)


def _fps(xyz, npoint):
    b, n, _ = xyz.shape
    xt = jnp.transpose(xyz, (0, 2, 1))
    return pl.pallas_call(
        functools.partial(_fps_kernel, npoint=npoint),
        in_specs=[pl.BlockSpec((b, 3, n), lambda: (0, 0, 0))],
        out_specs=pl.BlockSpec((b, npoint), lambda: (0, 0)),
        out_shape=jax.ShapeDtypeStruct((b, npoint), jnp.int32),
    )(xt)


def _write_stats(y, s1_ref, s2_ref):
    s1_ref[...] = jnp.broadcast_to(
        jnp.sum(y, axis=0, keepdims=True), s1_ref.shape
    )
    s2_ref[...] = jnp.broadcast_to(
        jnp.sum(y * y, axis=0, keepdims=True), s2_ref.shape
    )


def _mm_kernel(x_ref, w_ref, b_ref, o_ref, s1_ref, s2_ref):
    y = (
        jnp.dot(x_ref[...], w_ref[...].T, preferred_element_type=jnp.float32)
        + b_ref[...]
    )
    o_ref[...] = y
    _write_stats(y, s1_ref, s2_ref)


def _mm_act_kernel(x_ref, w_ref, b_ref, s_ref, t_ref, o_ref, s1_ref, s2_ref):
    x = jnp.maximum(x_ref[...] * s_ref[...] + t_ref[...], 0.0)
    y = jnp.dot(x, w_ref[...].T, preferred_element_type=jnp.float32) + b_ref[...]
    o_ref[...] = y
    _write_stats(y, s1_ref, s2_ref)


def _matmul(x, w, bvec, scale=None, shift=None):
    m, c = x.shape
    o = w.shape[0]
    tm = min(m, 4096)
    grid = (m // tm,)
    b2 = bvec.reshape(1, o)
    g = grid[0]
    x_spec = pl.BlockSpec((tm, c), lambda i: (i, 0))
    w_spec = pl.BlockSpec((o, c), lambda i: (0, 0))
    b_spec = pl.BlockSpec((1, o), lambda i: (0, 0))
    v_spec = pl.BlockSpec((1, c), lambda i: (0, 0))
    stat_spec = pl.BlockSpec((8, o), lambda i: (i, 0))
    out_specs = [
        pl.BlockSpec((tm, o), lambda i: (i, 0)),
        stat_spec,
        stat_spec,
    ]
    out_shape = [
        jax.ShapeDtypeStruct((m, o), jnp.float32),
        jax.ShapeDtypeStruct((g * 8, o), jnp.float32),
        jax.ShapeDtypeStruct((g * 8, o), jnp.float32),
    ]
    if scale is None:
        y, s1, s2 = pl.pallas_call(
            _mm_kernel,
            grid=grid,
            in_specs=[x_spec, w_spec, b_spec],
            out_specs=out_specs,
            out_shape=out_shape,
        )(x, w, b2)
    else:
        y, s1, s2 = pl.pallas_call(
            _mm_act_kernel,
            grid=grid,
            in_specs=[x_spec, w_spec, b_spec, v_spec, v_spec],
            out_specs=out_specs,
            out_shape=out_shape,
        )(x, w, b2, scale.reshape(1, c), shift.reshape(1, c))
    mean = jnp.sum(s1[::8], axis=0) / m
    var = jnp.sum(s2[::8], axis=0) / m - mean * mean
    return y, mean, var


def _arm_kernel(y_ref, s_ref, t_ref, o_ref):
    o_ref[...] = jnp.max(
        jnp.maximum(y_ref[...] * s_ref[...] + t_ref[...], 0.0), axis=1
    )


def _affine_relu_max(y, scale, shift):
    m2, k, o = y.shape
    ts = 8
    while ts * 2 <= m2 and m2 % (ts * 2) == 0 and ts * 2 * k * o * 4 <= 4 * 2**20:
        ts *= 2
    return pl.pallas_call(
        _arm_kernel,
        grid=(m2 // ts,),
        in_specs=[
            pl.BlockSpec((ts, k, o), lambda i: (i, 0, 0)),
            pl.BlockSpec((1, 1, o), lambda i: (0, 0, 0)),
            pl.BlockSpec((1, 1, o), lambda i: (0, 0, 0)),
        ],
        out_specs=pl.BlockSpec((ts, o), lambda i: (i, 0)),
        out_shape=jax.ShapeDtypeStruct((m2, o), jnp.float32),
    )(y, scale.reshape(1, 1, o), shift.reshape(1, 1, o))


def _bn_affine(mean, var, layer):
    scale = layer["gamma"] / jnp.sqrt(var + EPS)
    shift = layer["beta"] - mean * scale
    return scale, shift


def _mlp_max(grouped, layers):
    b, s, k, c = grouped.shape
    x = grouped.reshape(b * s * k, c)
    scale = shift = None
    for layer in layers:
        x, mean, var = _matmul(x, layer["W"], layer["b"], scale, shift)
        scale, shift = _bn_affine(mean, var, layer)
    o = x.shape[-1]
    out = _affine_relu_max(x.reshape(b * s, k, o), scale, shift)
    return out.reshape(b, s, o)


def _gather(points, idx):
    b = points.shape[0]
    batch_idx = jnp.arange(b).reshape((b,) + (1,) * (idx.ndim - 1))
    return points[batch_idx, idx]


def _ball_query(radius, nsample, xyz, new_xyz):
    b, n, _ = xyz.shape
    s = new_xyz.shape[1]
    sqrdists = (
        jnp.sum(new_xyz**2, axis=-1)[:, :, None]
        + jnp.sum(xyz**2, axis=-1)[:, None, :]
        - 2.0 * jnp.matmul(new_xyz, jnp.transpose(xyz, (0, 2, 1)))
    )
    group_idx = jnp.broadcast_to(jnp.arange(n, dtype=jnp.int32), (b, s, n))
    group_idx = jnp.where(sqrdists > radius**2, n, group_idx)
    group_idx = jnp.sort(group_idx, axis=-1)[:, :, :nsample]
    group_first = jnp.broadcast_to(group_idx[:, :, :1], group_idx.shape)
    return jnp.where(group_idx == n, group_first, group_idx)


def _sa_msg(xyz, points, npoint, radius_list, nsample_list, branches):
    fps_idx = _fps(xyz, npoint)
    new_xyz = _gather(xyz, fps_idx)
    outs = []
    for radius, nsample, layers in zip(radius_list, nsample_list, branches):
        group_idx = _ball_query(radius, nsample, xyz, new_xyz)
        grouped_xyz = _gather(xyz, group_idx) - new_xyz[:, :, None, :]
        if points is not None:
            grouped = jnp.concatenate(
                [_gather(points, group_idx), grouped_xyz], axis=-1
            )
        else:
            grouped = grouped_xyz
        outs.append(_mlp_max(grouped, layers))
    return new_xyz, jnp.concatenate(outs, axis=-1)


def kernel(xyz, params):
    pts = jnp.transpose(xyz, (0, 2, 1))
    l1_xyz, l1_points = _sa_msg(
        pts, None, 512, [0.1, 0.2, 0.4], [16, 32, 128], params["sa1"]
    )
    l2_xyz, l2_points = _sa_msg(
        l1_xyz, l1_points, 128, [0.2, 0.4, 0.8], [32, 64, 128], params["sa2"]
    )
    grouped = jnp.concatenate([l2_xyz, l2_points], axis=-1)[:, None, :, :]
    l3 = _mlp_max(grouped, params["sa3"])
    return l3.reshape(l3.shape[0], 1024)
